# Initial kernel scaffold; baseline (speedup 1.0000x reference)
import jax
import jax.numpy as jnp
from jax.experimental import pallas as pl


def kernel(pred_ins, pred_sem, true_ins, true_sem):
    raise NotImplementedError("write your pallas kernel here")


# speedup vs baseline: 10.8337x; 10.8337x over previous
import jax
import jax.numpy as jnp
from jax.experimental import pallas as pl
from jax.experimental.pallas import tpu as pltpu

DELTA_V = 0.5
DELTA_D = 1.5
P_VAR = 1.0
P_DIST = 1.0
P_REG = 0.001
NUM_CLASSES = 24
NUM_SEM = 13


def _loss_body(pred_ins_ref, pred_sem_ref, ins_lbl_ref, sem_lbl_ref, out_ref):
    i = pl.program_id(0)
    n = pred_ins_ref.shape[1]
    M = NUM_CLASSES

    pred = pred_ins_ref[0]
    lbl_col = ins_lbl_ref[0]

    cls_iota = jax.lax.broadcasted_iota(jnp.int32, (n, M), 1)
    oh = (lbl_col == cls_iota).astype(jnp.float32)

    seg = jax.lax.dot_general(
        oh, pred, (((0,), (0,)), ((), ())),
        preferred_element_type=jnp.float32)
    ones_col = jnp.ones((n, 1), dtype=jnp.float32)
    counts_col = jax.lax.dot_general(
        oh, ones_col, (((0,), (0,)), ((), ())),
        preferred_element_type=jnp.float32)

    present_col = counts_col > 0.0
    presentf_col = present_col.astype(jnp.float32)
    Kf = jnp.sum(presentf_col)
    cf_safe = jnp.where(present_col, counts_col, 1.0)
    mu = jnp.where(present_col, seg / cf_safe, 0.0)

    mu_exp = jnp.dot(oh, mu, preferred_element_type=jnp.float32)
    dist = jnp.sum(jnp.abs(pred - mu_exp), axis=1, keepdims=True)
    r = jnp.square(jnp.maximum(dist - DELTA_V, 0.0))
    segr = jax.lax.dot_general(
        oh, r, (((0,), (0,)), ((), ())),
        preferred_element_type=jnp.float32)
    l_var = jnp.sum(segr / cf_safe) / Kf

    diff = mu[:, None, :] - mu[None, :, :]
    n1 = jnp.sum(jnp.abs(diff), axis=2)
    mn = jnp.square(jnp.maximum(2.0 * DELTA_D - n1, 0.0))
    row_i = jax.lax.broadcasted_iota(jnp.int32, (M, M), 0)
    col_i = jax.lax.broadcasted_iota(jnp.int32, (M, M), 1)
    off_diag = (row_i != col_i).astype(jnp.float32)
    pair_mask = presentf_col * presentf_col.reshape(1, M) * off_diag
    denom = jnp.where(Kf > 1.0, Kf * (Kf - 1.0), 1.0)
    l_dist = jnp.where(Kf > 1.0, jnp.sum(mn * pair_mask) / denom, 0.0)

    l_reg = jnp.sum(jnp.abs(mu)) / Kf
    disc = P_VAR * l_var + P_DIST * l_dist + P_REG * l_reg

    x = pred_sem_ref[0]
    sem_col = sem_lbl_ref[0]
    m = jnp.max(x, axis=1, keepdims=True)
    lse = jnp.log(jnp.sum(jnp.exp(x - m), axis=1, keepdims=True)) + m
    sem_iota = jax.lax.broadcasted_iota(jnp.int32, (n, NUM_SEM), 1)
    oh_sem = (sem_col == sem_iota).astype(jnp.float32)
    xl = jnp.sum(x * oh_sem, axis=1, keepdims=True)
    nll_sum = jnp.sum(lse - xl)

    B = pl.num_programs(0)
    contrib = nll_sum / (B * n) + disc / B
    prev = jnp.where(i == 0, jnp.zeros((1, 1), jnp.float32), out_ref[...])
    out_ref[...] = prev + contrib


def kernel(pred_ins, pred_sem, true_ins, true_sem):
    B, n, D = pred_ins.shape
    ins_lbl = true_ins.reshape(B, n, 1)
    sem_lbl = true_sem.reshape(B, n, 1)

    loss = pl.pallas_call(
        _loss_body,
        grid=(B,),
        in_specs=[
            pl.BlockSpec((1, n, D), lambda i: (i, 0, 0)),
            pl.BlockSpec((1, n, NUM_SEM), lambda i: (i, 0, 0)),
            pl.BlockSpec((1, n, 1), lambda i: (i, 0, 0)),
            pl.BlockSpec((1, n, 1), lambda i: (i, 0, 0)),
        ],
        out_specs=pl.BlockSpec((1, 1), lambda i: (0, 0)),
        out_shape=jax.ShapeDtypeStruct((1, 1), jnp.float32),
    )(pred_ins, pred_sem, ins_lbl, sem_lbl)

    return (pred_sem, loss[0, 0])


# speedup vs baseline: 37.1085x; 3.4253x over previous
import jax
import jax.numpy as jnp
from jax.experimental import pallas as pl
from jax.experimental.pallas import tpu as pltpu

DELTA_V = 0.5
DELTA_D = 1.5
P_VAR = 1.0
P_DIST = 1.0
P_REG = 0.001
NUM_CLASSES = 24
NUM_SEM = 13


def _loss_body(pred_ins_ref, pred_sem_ref, ins_lbl_ref, sem_lbl_ref, out_ref):
    i = pl.program_id(0)
    n = pred_ins_ref.shape[1]
    M = NUM_CLASSES

    pred = pred_ins_ref[0]
    lbl_row = ins_lbl_ref[0]

    cls_iota = jax.lax.broadcasted_iota(jnp.int32, (M, n), 0)
    oh_t = (cls_iota == lbl_row).astype(jnp.float32)

    seg = jnp.dot(oh_t, pred, preferred_element_type=jnp.float32)
    counts_col = jnp.sum(oh_t, axis=1, keepdims=True)

    present_col = counts_col > 0.0
    presentf_col = present_col.astype(jnp.float32)
    Kf = jnp.sum(presentf_col)
    cf_safe = jnp.where(present_col, counts_col, 1.0)
    mu = jnp.where(present_col, seg / cf_safe, 0.0)

    mu_exp = jax.lax.dot_general(
        oh_t, mu, (((0,), (0,)), ((), ())),
        preferred_element_type=jnp.float32)
    dist = jnp.sum(jnp.abs(pred - mu_exp), axis=1, keepdims=True)
    r = jnp.square(jnp.maximum(dist - DELTA_V, 0.0))
    segr = jnp.dot(oh_t, r, preferred_element_type=jnp.float32)
    l_var = jnp.sum(segr / cf_safe) / Kf

    diff = mu[:, None, :] - mu[None, :, :]
    n1 = jnp.sum(jnp.abs(diff), axis=2)
    mn = jnp.square(jnp.maximum(2.0 * DELTA_D - n1, 0.0))
    row_i = jax.lax.broadcasted_iota(jnp.int32, (M, M), 0)
    col_i = jax.lax.broadcasted_iota(jnp.int32, (M, M), 1)
    off_diag = (row_i != col_i).astype(jnp.float32)
    pair_mask = presentf_col * presentf_col.reshape(1, M) * off_diag
    denom = jnp.where(Kf > 1.0, Kf * (Kf - 1.0), 1.0)
    l_dist = jnp.where(Kf > 1.0, jnp.sum(mn * pair_mask) / denom, 0.0)

    l_reg = jnp.sum(jnp.abs(mu)) / Kf
    disc = P_VAR * l_var + P_DIST * l_dist + P_REG * l_reg

    x = pred_sem_ref[0]
    sem_row = sem_lbl_ref[0]
    m = jnp.max(x, axis=0, keepdims=True)
    lse = jnp.log(jnp.sum(jnp.exp(x - m), axis=0, keepdims=True)) + m
    sem_iota = jax.lax.broadcasted_iota(jnp.int32, (NUM_SEM, n), 0)
    oh_sem_t = (sem_iota == sem_row).astype(jnp.float32)
    xl = jnp.sum(x * oh_sem_t, axis=0, keepdims=True)
    nll_sum = jnp.sum(lse - xl)

    B = pl.num_programs(0)
    contrib = nll_sum / (B * n) + disc / B
    prev = jnp.where(i == 0, jnp.zeros((1, 1), jnp.float32), out_ref[...])
    out_ref[...] = prev + contrib


def kernel(pred_ins, pred_sem, true_ins, true_sem):
    B, n, D = pred_ins.shape
    sem_t = jnp.transpose(pred_sem, (0, 2, 1))
    ins_lbl = true_ins.reshape(B, 1, n)
    sem_lbl = true_sem.reshape(B, 1, n)

    loss = pl.pallas_call(
        _loss_body,
        grid=(B,),
        in_specs=[
            pl.BlockSpec((1, n, D), lambda i: (i, 0, 0)),
            pl.BlockSpec((1, NUM_SEM, n), lambda i: (i, 0, 0)),
            pl.BlockSpec((1, 1, n), lambda i: (i, 0, 0)),
            pl.BlockSpec((1, 1, n), lambda i: (i, 0, 0)),
        ],
        out_specs=pl.BlockSpec((1, 1), lambda i: (0, 0)),
        out_shape=jax.ShapeDtypeStruct((1, 1), jnp.float32),
    )(pred_ins, sem_t, ins_lbl, sem_lbl)

    return (pred_sem, loss[0, 0])
